# baseline (speedup 1.0000x reference)
import jax
import jax.numpy as jnp
from jax import lax
from jax.experimental import pallas as pl
from jax.experimental.pallas import tpu as pltpu
from jax.experimental.pallas import tpu_sc as plsc

_NUM_USERS = 1000
_VOCAB = 2000
_D = 128
_B = 4096
_NW = 32
_BPW = _B // _NW


def _v_body(ww, wfw, out):
    out[...] = jnp.dot(ww[...], wfw[...], preferred_element_type=jnp.float32)


def _wide_v(W_wide, Wf_w):
    return pl.pallas_call(
        _v_body,
        out_shape=jax.ShapeDtypeStruct((_VOCAB, 1), jnp.float32),
    )(W_wide, Wf_w)


def _sc_gather_body(user_hbm, item_hbm, emb2_hbm, v_hbm,
                    emb_out, wv_out,
                    uidx, iidx, gu_v, gi_v, v_v, wv_v, sem):
    wid = lax.axis_index("s") * 2 + lax.axis_index("c")
    base = wid * _BPW
    pltpu.sync_copy(user_hbm.at[pl.ds(base, _BPW)], uidx)
    pltpu.sync_copy(item_hbm.at[pl.ds(base, _BPW)], iidx)
    for j in range(_BPW // 16):
        iidx[pl.ds(j * 16, 16)] = iidx[pl.ds(j * 16, 16)] + _NUM_USERS
    c0 = pltpu.async_copy(emb2_hbm.at[uidx], gu_v, sem)
    c1 = pltpu.async_copy(emb2_hbm.at[iidx], gi_v, sem)
    pltpu.sync_copy(v_hbm, v_v)
    for j in range(_BPW // 16):
        vg_u = plsc.load_gather(v_v, [uidx[pl.ds(j * 16, 16)]])
        vg_i = plsc.load_gather(v_v, [iidx[pl.ds(j * 16, 16)]])
        wv_v[pl.ds(j * 16, 16)] = vg_u + vg_i
    c0.wait()
    c1.wait()

    def _add_row(i, carry):
        for j in range(_D // 16):
            sl = pl.ds(j * 16, 16)
            gu_v[i, sl] = gu_v[i, sl] + gi_v[i, sl]
        return carry

    lax.fori_loop(0, _BPW, _add_row, 0)
    pltpu.sync_copy(gu_v, emb_out.at[pl.ds(base, _BPW)])
    pltpu.sync_copy(wv_v, wv_out.at[pl.ds(base, _BPW)])


def _sc_gather(user, item, emb2, v):
    mesh = plsc.VectorSubcoreMesh(core_axis_name="c", subcore_axis_name="s")
    f = pl.kernel(
        _sc_gather_body, mesh=mesh,
        compiler_params=pltpu.CompilerParams(needs_layout_passes=False),
        out_type=(
            jax.ShapeDtypeStruct((_B, _D), jnp.float32),
            jax.ShapeDtypeStruct((_B,), jnp.float32),
        ),
        scratch_types=[
            pltpu.VMEM((_BPW,), jnp.int32),
            pltpu.VMEM((_BPW,), jnp.int32),
            pltpu.VMEM((_BPW, _D), jnp.float32),
            pltpu.VMEM((_BPW, _D), jnp.float32),
            pltpu.VMEM((_VOCAB,), jnp.float32),
            pltpu.VMEM((_BPW,), jnp.float32),
            pltpu.SemaphoreType.DMA,
        ],
    )
    return f(user, item, emb2, v)


def _mlp_body(emb, g, t, wv, W0, b0, W1, b1, Wf, bf, b_wide, out):
    x = (jnp.dot(emb[...], W0[0:128, :], preferred_element_type=jnp.float32)
         + jnp.dot(g[...], W0[128:148, :], preferred_element_type=jnp.float32)
         + jnp.dot(t[...], W0[148:248, :], preferred_element_type=jnp.float32)
         + b0[...])
    h0 = jnp.maximum(x, 0.0)
    h1 = jnp.maximum(
        jnp.dot(h0, W1[...], preferred_element_type=jnp.float32) + b1[...], 0.0)
    wide_bias = jnp.sum(b_wide[...] * Wf[128:256, 0]) + bf[0]
    logits = (jnp.dot(h1, Wf[0:128, :], preferred_element_type=jnp.float32)
              + wv[...] + wide_bias)
    out[...] = logits


def _mlp(emb, genre, tag, wv, W0, b0, W1, b1, Wf, bf, b_wide):
    nb = 4
    blk = _B // nb
    rep = lambda shape: pl.BlockSpec(shape, lambda i: (0,) * len(shape))
    row = lambda d: pl.BlockSpec((blk, d), lambda i: (i, 0))
    return pl.pallas_call(
        _mlp_body,
        grid=(nb,),
        in_specs=[
            row(_D), row(20), row(100), row(1),
            rep((248, 256)), rep((256,)), rep((256, 128)), rep((128,)),
            rep((256, 1)), rep((1,)), rep((128,)),
        ],
        out_specs=row(1),
        out_shape=jax.ShapeDtypeStruct((_B, 1), jnp.float32),
    )(emb, genre, tag, wv, W0, b0, W1, b1, Wf, bf, b_wide)


def kernel(user, item, genre, tag, W_wide, b_wide, user_table, item_table,
           W0, b0, W1, b1, Wf, bf):
    user = user.astype(jnp.int32)
    item = item.astype(jnp.int32)
    zeros = jnp.zeros_like(user_table)
    emb2 = jnp.concatenate([
        jnp.concatenate([user_table, zeros], axis=1),
        jnp.concatenate([zeros, item_table], axis=1),
    ], axis=0)
    v = _wide_v(W_wide, Wf[128:256, :])[:, 0]
    emb, wv = _sc_gather(user, item, emb2, v)
    return _mlp(emb, genre, tag, wv.reshape(_B, 1),
                W0, b0, W1, b1, Wf, bf, b_wide)


# speedup vs baseline: 1.0251x; 1.0251x over previous
import jax
import jax.numpy as jnp
from jax import lax
from jax.experimental import pallas as pl
from jax.experimental.pallas import tpu as pltpu
from jax.experimental.pallas import tpu_sc as plsc

_NUM_USERS = 1000
_D = 128
_B = 4096
_NW = 32
_BPW = _B // _NW


def _sc_gather_body(user_hbm, item_hbm, u2_hbm, i2_hbm, ww_hbm,
                    gu_out, gi_out, w_out,
                    uidx, iidx, uh, ih, gu_v, gi_v, wu_v, wi_v, sem):
    wid = lax.axis_index("s") * 2 + lax.axis_index("c")
    base = wid * _BPW
    pltpu.sync_copy(user_hbm.at[pl.ds(base, _BPW)], uidx)
    pltpu.sync_copy(item_hbm.at[pl.ds(base, _BPW)], iidx)
    for j in range(_BPW // 16):
        sl = pl.ds(j * 16, 16)
        u = uidx[sl]
        i = iidx[sl]
        uh[sl] = lax.shift_right_logical(u, 1)
        ih[sl] = lax.shift_right_logical(i, 1)
        iidx[sl] = i + _NUM_USERS
    c0 = pltpu.async_copy(u2_hbm.at[uh], gu_v, sem)
    c1 = pltpu.async_copy(i2_hbm.at[ih], gi_v, sem)
    c2 = pltpu.async_copy(ww_hbm.at[uidx], wu_v, sem)
    c3 = pltpu.async_copy(ww_hbm.at[iidx], wi_v, sem)
    c0.wait()
    c1.wait()
    c2.wait()
    c3.wait()

    def _add_row(i, carry):
        for j in range(_D // 16):
            sl = pl.ds(j * 16, 16)
            wu_v[i, sl] = wu_v[i, sl] + wi_v[i, sl]
        return carry

    lax.fori_loop(0, _BPW, _add_row, 0)
    pltpu.sync_copy(gu_v, gu_out.at[pl.ds(base, _BPW)])
    pltpu.sync_copy(gi_v, gi_out.at[pl.ds(base, _BPW)])
    pltpu.sync_copy(wu_v, w_out.at[pl.ds(base, _BPW)])


def _sc_gather(user, item, u2, i2, W_wide):
    mesh = plsc.VectorSubcoreMesh(core_axis_name="c", subcore_axis_name="s")
    f = pl.kernel(
        _sc_gather_body, mesh=mesh,
        compiler_params=pltpu.CompilerParams(needs_layout_passes=False),
        out_type=tuple(
            jax.ShapeDtypeStruct((_B, _D), jnp.float32) for _ in range(3)),
        scratch_types=[
            pltpu.VMEM((_BPW,), jnp.int32),
            pltpu.VMEM((_BPW,), jnp.int32),
            pltpu.VMEM((_BPW,), jnp.int32),
            pltpu.VMEM((_BPW,), jnp.int32),
            pltpu.VMEM((_BPW, _D), jnp.float32),
            pltpu.VMEM((_BPW, _D), jnp.float32),
            pltpu.VMEM((_BPW, _D), jnp.float32),
            pltpu.VMEM((_BPW, _D), jnp.float32),
            pltpu.SemaphoreType.DMA,
        ],
    )
    return f(user, item, u2, i2, W_wide)


def _mlp_body(u2d, i2d, gu, gi, g, t, w, W0, b0, W1, b1, Wf, bf, b_wide, out):
    um = (u2d[...] & 1) == 1
    im = (i2d[...] & 1) == 1
    guv = gu[...]
    giv = gi[...]
    ue = jnp.where(um, guv[:, 64:128], guv[:, 0:64])
    ie = jnp.where(im, giv[:, 64:128], giv[:, 0:64])
    x = (jnp.dot(ue, W0[0:64, :], preferred_element_type=jnp.float32)
         + jnp.dot(ie, W0[64:128, :], preferred_element_type=jnp.float32)
         + jnp.dot(g[...], W0[128:148, :], preferred_element_type=jnp.float32)
         + jnp.dot(t[...], W0[148:248, :], preferred_element_type=jnp.float32)
         + b0[...])
    h0 = jnp.maximum(x, 0.0)
    h1 = jnp.maximum(
        jnp.dot(h0, W1[...], preferred_element_type=jnp.float32) + b1[...], 0.0)
    wide = w[...] + b_wide[...]
    logits = (jnp.dot(h1, Wf[0:128, :], preferred_element_type=jnp.float32)
              + jnp.dot(wide, Wf[128:256, :], preferred_element_type=jnp.float32)
              + bf[...])
    out[...] = logits


def _mlp(u2d, i2d, gu, gi, genre, tag, w, W0, b0, W1, b1, Wf, bf, b_wide):
    nb = 4
    blk = _B // nb
    rep = lambda shape: pl.BlockSpec(shape, lambda i: (0,) * len(shape))
    row = lambda d: pl.BlockSpec((blk, d), lambda i: (i, 0))
    return pl.pallas_call(
        _mlp_body,
        grid=(nb,),
        in_specs=[
            row(1), row(1), row(_D), row(_D), row(20), row(100), row(_D),
            rep((248, 256)), rep((256,)), rep((256, 128)), rep((128,)),
            rep((256, 1)), rep((1,)), rep((128,)),
        ],
        out_specs=row(1),
        out_shape=jax.ShapeDtypeStruct((_B, 1), jnp.float32),
    )(u2d, i2d, gu, gi, genre, tag, w, W0, b0, W1, b1, Wf, bf, b_wide)


def kernel(user, item, genre, tag, W_wide, b_wide, user_table, item_table,
           W0, b0, W1, b1, Wf, bf):
    user = user.astype(jnp.int32)
    item = item.astype(jnp.int32)
    u2 = user_table.reshape(_NUM_USERS // 2, _D)
    i2 = item_table.reshape(_NUM_USERS // 2, _D)
    gu, gi, w = _sc_gather(user, item, u2, i2, W_wide)
    return _mlp(user.reshape(_B, 1), item.reshape(_B, 1), gu, gi, genre, tag,
                w, W0, b0, W1, b1, Wf, bf, b_wide)


# speedup vs baseline: 1.1204x; 1.0930x over previous
import jax
import jax.numpy as jnp
from jax import lax
from jax.experimental import pallas as pl
from jax.experimental.pallas import tpu as pltpu
from jax.experimental.pallas import tpu_sc as plsc

_NUM_USERS = 1000
_D = 128
_B = 4096
_NW = 32
_BPW = _B // _NW


def _sc_gather_body(user_hbm, item_hbm, emb2_hbm, ww_hbm,
                    gu_out, gi_out, wu_out, wi_out,
                    uidx, iidx, gu_v, gi_v, wu_v, wi_v, sem_i, sem_g, sem_w):
    wid = lax.axis_index("s") * 2 + lax.axis_index("c")
    base = wid * _BPW
    ci0 = pltpu.async_copy(user_hbm.at[pl.ds(base, _BPW)], uidx, sem_i)
    ci1 = pltpu.async_copy(item_hbm.at[pl.ds(base, _BPW)], iidx, sem_i)
    ci0.wait()
    ci1.wait()
    for j in range(_BPW // 16):
        iidx[pl.ds(j * 16, 16)] = iidx[pl.ds(j * 16, 16)] + _NUM_USERS
    c0 = pltpu.async_copy(emb2_hbm.at[uidx], gu_v, sem_g)
    c1 = pltpu.async_copy(emb2_hbm.at[iidx], gi_v, sem_g)
    c2 = pltpu.async_copy(ww_hbm.at[uidx], wu_v, sem_g)
    c3 = pltpu.async_copy(ww_hbm.at[iidx], wi_v, sem_g)
    c0.wait()
    w0 = pltpu.async_copy(gu_v, gu_out.at[pl.ds(base, _BPW)], sem_w)
    c1.wait()
    w1 = pltpu.async_copy(gi_v, gi_out.at[pl.ds(base, _BPW)], sem_w)
    c2.wait()
    w2 = pltpu.async_copy(wu_v, wu_out.at[pl.ds(base, _BPW)], sem_w)
    c3.wait()
    w3 = pltpu.async_copy(wi_v, wi_out.at[pl.ds(base, _BPW)], sem_w)
    w0.wait()
    w1.wait()
    w2.wait()
    w3.wait()


def _sc_gather(user, item, emb2, W_wide):
    mesh = plsc.VectorSubcoreMesh(core_axis_name="c", subcore_axis_name="s")
    f = pl.kernel(
        _sc_gather_body, mesh=mesh,
        compiler_params=pltpu.CompilerParams(needs_layout_passes=False),
        out_type=tuple(
            jax.ShapeDtypeStruct((_B, _D), jnp.float32) for _ in range(4)),
        scratch_types=[
            pltpu.VMEM((_BPW,), jnp.int32),
            pltpu.VMEM((_BPW,), jnp.int32),
            pltpu.VMEM((_BPW, _D), jnp.float32),
            pltpu.VMEM((_BPW, _D), jnp.float32),
            pltpu.VMEM((_BPW, _D), jnp.float32),
            pltpu.VMEM((_BPW, _D), jnp.float32),
            pltpu.SemaphoreType.DMA,
            pltpu.SemaphoreType.DMA,
            pltpu.SemaphoreType.DMA,
        ],
    )
    return f(user, item, emb2, W_wide)


def _mlp_body(gu, gi, g, t, wu, wi, W0, b0, W1, b1, Wf, bf, b_wide, out):
    emb = gu[...] + gi[...]
    x = (jnp.dot(emb, W0[0:128, :], preferred_element_type=jnp.float32)
         + jnp.dot(g[...], W0[128:148, :], preferred_element_type=jnp.float32)
         + jnp.dot(t[...], W0[148:248, :], preferred_element_type=jnp.float32)
         + b0[...])
    h0 = jnp.maximum(x, 0.0)
    h1 = jnp.maximum(
        jnp.dot(h0, W1[...], preferred_element_type=jnp.float32) + b1[...], 0.0)
    wide = wu[...] + wi[...] + b_wide[...]
    logits = (jnp.dot(h1, Wf[0:128, :], preferred_element_type=jnp.float32)
              + jnp.dot(wide, Wf[128:256, :], preferred_element_type=jnp.float32)
              + bf[...])
    out[...] = logits


def _mlp(gu, gi, genre, tag, wu, wi, W0, b0, W1, b1, Wf, bf, b_wide):
    nb = 4
    blk = _B // nb
    rep = lambda shape: pl.BlockSpec(shape, lambda i: (0,) * len(shape))
    row = lambda d: pl.BlockSpec((blk, d), lambda i: (i, 0))
    return pl.pallas_call(
        _mlp_body,
        grid=(nb,),
        in_specs=[
            row(_D), row(_D), row(20), row(100), row(_D), row(_D),
            rep((248, 256)), rep((256,)), rep((256, 128)), rep((128,)),
            rep((256, 1)), rep((1,)), rep((128,)),
        ],
        out_specs=row(1),
        out_shape=jax.ShapeDtypeStruct((_B, 1), jnp.float32),
    )(gu, gi, genre, tag, wu, wi, W0, b0, W1, b1, Wf, bf, b_wide)


def kernel(user, item, genre, tag, W_wide, b_wide, user_table, item_table,
           W0, b0, W1, b1, Wf, bf):
    user = user.astype(jnp.int32)
    item = item.astype(jnp.int32)
    zeros = jnp.zeros_like(user_table)
    emb2 = jnp.concatenate([
        jnp.concatenate([user_table, zeros], axis=1),
        jnp.concatenate([zeros, item_table], axis=1),
    ], axis=0)
    gu, gi, wu, wi = _sc_gather(user, item, emb2, W_wide)
    return _mlp(gu, gi, genre, tag, wu, wi, W0, b0, W1, b1, Wf, bf, b_wide)


# speedup vs baseline: 1.1209x; 1.0005x over previous
import jax
import jax.numpy as jnp
from jax import lax
from jax.experimental import pallas as pl
from jax.experimental.pallas import tpu as pltpu
from jax.experimental.pallas import tpu_sc as plsc

_NUM_USERS = 1000
_D = 128
_B = 4096
_NW = 32
_BPW = _B // _NW


def _sc_gather_body(user_hbm, item_hbm, emb2_hbm, ww_hbm,
                    gu_out, gi_out, wu_out, wi_out,
                    uidx, iidx, gu_v, gi_v, wu_v, wi_v, sem_i, sem_g, sem_w):
    wid = lax.axis_index("s") * 2 + lax.axis_index("c")
    base = wid * _BPW
    ci0 = pltpu.async_copy(user_hbm.at[pl.ds(base, _BPW)], uidx, sem_i)
    ci1 = pltpu.async_copy(item_hbm.at[pl.ds(base, _BPW)], iidx, sem_i)
    ci0.wait()
    ci1.wait()
    for j in range(_BPW // 16):
        iidx[pl.ds(j * 16, 16)] = iidx[pl.ds(j * 16, 16)] + _NUM_USERS
    c0 = pltpu.async_copy(emb2_hbm.at[uidx], gu_v, sem_g)
    c1 = pltpu.async_copy(emb2_hbm.at[iidx], gi_v, sem_g)
    c2 = pltpu.async_copy(ww_hbm.at[uidx], wu_v, sem_g)
    c3 = pltpu.async_copy(ww_hbm.at[iidx], wi_v, sem_g)
    c0.wait()
    w0 = pltpu.async_copy(gu_v, gu_out.at[pl.ds(base, _BPW)], sem_w)
    c1.wait()
    w1 = pltpu.async_copy(gi_v, gi_out.at[pl.ds(base, _BPW)], sem_w)
    c2.wait()
    w2 = pltpu.async_copy(wu_v, wu_out.at[pl.ds(base, _BPW)], sem_w)
    c3.wait()
    w3 = pltpu.async_copy(wi_v, wi_out.at[pl.ds(base, _BPW)], sem_w)
    w0.wait()
    w1.wait()
    w2.wait()
    w3.wait()


def _sc_gather(user, item, emb2, W_wide):
    mesh = plsc.VectorSubcoreMesh(core_axis_name="c", subcore_axis_name="s")
    f = pl.kernel(
        _sc_gather_body, mesh=mesh,
        compiler_params=pltpu.CompilerParams(
            needs_layout_passes=False, skip_device_barrier=True),
        out_type=tuple(
            jax.ShapeDtypeStruct((_B, _D), jnp.float32) for _ in range(4)),
        scratch_types=[
            pltpu.VMEM((_BPW,), jnp.int32),
            pltpu.VMEM((_BPW,), jnp.int32),
            pltpu.VMEM((_BPW, _D), jnp.float32),
            pltpu.VMEM((_BPW, _D), jnp.float32),
            pltpu.VMEM((_BPW, _D), jnp.float32),
            pltpu.VMEM((_BPW, _D), jnp.float32),
            pltpu.SemaphoreType.DMA,
            pltpu.SemaphoreType.DMA,
            pltpu.SemaphoreType.DMA,
        ],
    )
    return f(user, item, emb2, W_wide)


def _mlp_body(gu, gi, g, t, wu, wi, W0, b0, W1, b1, Wf, bf, b_wide, out):
    emb = gu[...] + gi[...]
    x = (jnp.dot(emb, W0[0:128, :], preferred_element_type=jnp.float32)
         + jnp.dot(g[...], W0[128:148, :], preferred_element_type=jnp.float32)
         + jnp.dot(t[...], W0[148:248, :], preferred_element_type=jnp.float32)
         + b0[...])
    h0 = jnp.maximum(x, 0.0)
    h1 = jnp.maximum(
        jnp.dot(h0, W1[...], preferred_element_type=jnp.float32) + b1[...], 0.0)
    wide = wu[...] + wi[...] + b_wide[...]
    logits = (jnp.dot(h1, Wf[0:128, :], preferred_element_type=jnp.float32)
              + jnp.dot(wide, Wf[128:256, :], preferred_element_type=jnp.float32)
              + bf[...])
    out[...] = logits


def _mlp(gu, gi, genre, tag, wu, wi, W0, b0, W1, b1, Wf, bf, b_wide):
    nb = 4
    blk = _B // nb
    rep = lambda shape: pl.BlockSpec(shape, lambda i: (0,) * len(shape))
    row = lambda d: pl.BlockSpec((blk, d), lambda i: (i, 0))
    return pl.pallas_call(
        _mlp_body,
        grid=(nb,),
        compiler_params=pltpu.CompilerParams(skip_device_barrier=True),
        in_specs=[
            row(_D), row(_D), row(20), row(100), row(_D), row(_D),
            rep((248, 256)), rep((256,)), rep((256, 128)), rep((128,)),
            rep((256, 1)), rep((1,)), rep((128,)),
        ],
        out_specs=row(1),
        out_shape=jax.ShapeDtypeStruct((_B, 1), jnp.float32),
    )(gu, gi, genre, tag, wu, wi, W0, b0, W1, b1, Wf, bf, b_wide)


def kernel(user, item, genre, tag, W_wide, b_wide, user_table, item_table,
           W0, b0, W1, b1, Wf, bf):
    user = user.astype(jnp.int32)
    item = item.astype(jnp.int32)
    zeros = jnp.zeros_like(user_table)
    emb2 = jnp.concatenate([
        jnp.concatenate([user_table, zeros], axis=1),
        jnp.concatenate([zeros, item_table], axis=1),
    ], axis=0)
    gu, gi, wu, wi = _sc_gather(user, item, emb2, W_wide)
    return _mlp(gu, gi, genre, tag, wu, wi, W0, b0, W1, b1, Wf, bf, b_wide)
